# baseline (speedup 1.0000x reference)
import jax
import jax.numpy as jnp
from jax import lax
from jax.experimental import pallas as pl
from jax.experimental.pallas import tpu as pltpu
from jax.experimental.pallas import tpu_sc as plsc

N_USERS = 100000
N_ITEMS = 100000
D = 128
B = 16384
L = 16
NC = 2
NS = 16
NW = NC * NS
BW = B // NW
C = 128
NG = BW // C


def _body(uidx_hbm, pidx_hbm, nidx_hbm, utab_hbm, itab_hbm,
          pos_out, neg_out, reg_out,
          idxu_v, idxp_v, idxn_v,
          ru0, rp0, rn0, ru1, rp1, rn1,
          ps_v, ns_v, reg_v, isem, sem0, sem1):
    wid = lax.axis_index("s") * NC + lax.axis_index("c")
    base = wid * BW

    ci = (pltpu.async_copy(uidx_hbm.at[pl.ds(base, BW)], idxu_v, isem),
          pltpu.async_copy(pidx_hbm.at[pl.ds(base, BW)], idxp_v, isem),
          pltpu.async_copy(nidx_hbm.at[pl.ds(base, BW)], idxn_v, isem))
    for cp in ci:
        cp.wait()

    bufs = ((ru0, rp0, rn0, sem0), (ru1, rp1, rn1, sem1))

    def issue(g):
        ru, rp, rn, sem = bufs[g % 2]
        s = pl.ds(g * C, C)
        return (pltpu.async_copy(utab_hbm.at[idxu_v.at[s]], ru, sem),
                pltpu.async_copy(itab_hbm.at[idxp_v.at[s]], rp, sem),
                pltpu.async_copy(itab_hbm.at[idxn_v.at[s]], rn, sem))

    zf = jnp.zeros((L,), jnp.float32)
    reg_acc = zf
    pending = issue(0)

    for g in range(NG):
        nxt = issue(g + 1) if g + 1 < NG else None
        for cp in pending:
            cp.wait()
        pending = nxt
        ru, rp, rn, _ = bufs[g % 2]

        def tbody(t, racc, ru=ru, rp=rp, rn=rn, g=g):
            row_ids = lax.iota(jnp.int32, L) + t * L
            ap0 = ap1 = an0 = an1 = ar0 = ar1 = zf
            lane = lax.iota(jnp.int32, L)
            for d in range(D):
                col = (lane * 9 + d) & (D - 1)
                u = plsc.load_gather(ru, [row_ids, col])
                p = plsc.load_gather(rp, [row_ids, col])
                n = plsc.load_gather(rn, [row_ids, col])
                sq = u * u + (p * p + n * n)
                if d % 2 == 0:
                    ap0 = ap0 + u * p
                    an0 = an0 + u * n
                    ar0 = ar0 + sq
                else:
                    ap1 = ap1 + u * p
                    an1 = an1 + u * n
                    ar1 = ar1 + sq
            off = g * C + t * L
            ps_v[pl.ds(off, L)] = ap0 + ap1
            ns_v[pl.ds(off, L)] = an0 + an1
            return racc + (ar0 + ar1)

        reg_acc = lax.fori_loop(0, C // L, tbody, reg_acc)

    reg_v[...] = reg_acc * (1.0 / B)
    pltpu.sync_copy(ps_v, pos_out.at[pl.ds(base, BW)])
    pltpu.sync_copy(ns_v, neg_out.at[pl.ds(base, BW)])
    pltpu.sync_copy(reg_v, reg_out.at[wid])


_mesh = plsc.VectorSubcoreMesh(core_axis_name="c", subcore_axis_name="s")

_sc_call = pl.kernel(
    _body,
    out_type=[
        jax.ShapeDtypeStruct((B,), jnp.float32),
        jax.ShapeDtypeStruct((B,), jnp.float32),
        jax.ShapeDtypeStruct((NW, L), jnp.float32),
    ],
    mesh=_mesh,
    compiler_params=pltpu.CompilerParams(needs_layout_passes=False),
    scratch_types=[
        pltpu.VMEM((BW,), jnp.int32),
        pltpu.VMEM((BW,), jnp.int32),
        pltpu.VMEM((BW,), jnp.int32),
        pltpu.VMEM((C, D), jnp.float32),
        pltpu.VMEM((C, D), jnp.float32),
        pltpu.VMEM((C, D), jnp.float32),
        pltpu.VMEM((C, D), jnp.float32),
        pltpu.VMEM((C, D), jnp.float32),
        pltpu.VMEM((C, D), jnp.float32),
        pltpu.VMEM((BW,), jnp.float32),
        pltpu.VMEM((BW,), jnp.float32),
        pltpu.VMEM((L,), jnp.float32),
        pltpu.SemaphoreType.DMA,
        pltpu.SemaphoreType.DMA,
        pltpu.SemaphoreType.DMA,
    ],
)


def kernel(user_idx, pos_item, neg_item, user_table, item_table):
    ui = user_idx.astype(jnp.int32)
    pi = pos_item.astype(jnp.int32)
    ni = neg_item.astype(jnp.int32)
    pos_scores, neg_scores, reg_part = _sc_call(ui, pi, ni, user_table, item_table)
    return pos_scores, neg_scores, jnp.sum(reg_part)


# speedup vs baseline: 1.2071x; 1.2071x over previous
import jax
import jax.numpy as jnp
from jax import lax
from jax.experimental import pallas as pl
from jax.experimental.pallas import tpu as pltpu
from jax.experimental.pallas import tpu_sc as plsc

N_USERS = 100000
N_ITEMS = 100000
D = 128
B = 16384
L = 16
NC = 2
NS = 16
NW = NC * NS
BW = B // NW
C = 128
NG = BW // C
DK = D // L


def _body(uidx_hbm, pidx_hbm, nidx_hbm, utab_hbm, itab_hbm,
          pos_out, neg_out, reg_out,
          idxu_v, idxp_v, idxn_v,
          ru0, rp0, rn0, ru1, rp1, rn1,
          trp_v, trn_v, ps_v, ns_v, reg_v, isem, sem0, sem1):
    wid = lax.axis_index("s") * NC + lax.axis_index("c")
    base = wid * BW

    ci = (pltpu.async_copy(uidx_hbm.at[pl.ds(base, BW)], idxu_v, isem),
          pltpu.async_copy(pidx_hbm.at[pl.ds(base, BW)], idxp_v, isem),
          pltpu.async_copy(nidx_hbm.at[pl.ds(base, BW)], idxn_v, isem))
    for cp in ci:
        cp.wait()

    bufs = ((ru0, rp0, rn0, sem0), (ru1, rp1, rn1, sem1))

    def issue(g):
        ru, rp, rn, sem = bufs[g % 2]
        s = pl.ds(g * C, C)
        return (pltpu.async_copy(utab_hbm.at[idxu_v.at[s]], ru, sem),
                pltpu.async_copy(itab_hbm.at[idxp_v.at[s]], rp, sem),
                pltpu.async_copy(itab_hbm.at[idxn_v.at[s]], rn, sem))

    zf = jnp.zeros((L,), jnp.float32)
    lane = lax.iota(jnp.int32, L)
    reg_acc = zf
    pending = issue(0)

    for g in range(NG):
        nxt = issue(g + 1) if g + 1 < NG else None
        for cp in pending:
            cp.wait()
        pending = nxt
        ru, rp, rn, _ = bufs[g % 2]

        def tbody(t, racc, ru=ru, rp=rp, rn=rn, g=g):
            r0 = t * L
            ap = [zf] * L
            an = [zf] * L
            rr = [zf] * 4
            for j in range(L):
                for k in range(DK):
                    u = ru[r0 + j, pl.ds(k * L, L)]
                    p = rp[r0 + j, pl.ds(k * L, L)]
                    n = rn[r0 + j, pl.ds(k * L, L)]
                    ap[j] = ap[j] + u * p
                    an[j] = an[j] + u * n
                    rr[k % 4] = rr[k % 4] + (u * u + (p * p + n * n))
            for j in range(L):
                trp_v[j, pl.ds(0, L)] = ap[j]
                trn_v[j, pl.ds(0, L)] = an[j]
            sp = zf
            sn = zf
            for k in range(L):
                dia = (lane + k) & (L - 1)
                sp = sp + plsc.load_gather(trp_v, [lane, dia])
                sn = sn + plsc.load_gather(trn_v, [lane, dia])
            off = g * C + t * L
            ps_v[pl.ds(off, L)] = sp
            ns_v[pl.ds(off, L)] = sn
            return racc + ((rr[0] + rr[1]) + (rr[2] + rr[3]))

        reg_acc = lax.fori_loop(0, C // L, tbody, reg_acc)

    reg_v[...] = reg_acc * (1.0 / B)
    pltpu.sync_copy(ps_v, pos_out.at[pl.ds(base, BW)])
    pltpu.sync_copy(ns_v, neg_out.at[pl.ds(base, BW)])
    pltpu.sync_copy(reg_v, reg_out.at[wid])


_mesh = plsc.VectorSubcoreMesh(core_axis_name="c", subcore_axis_name="s")

_sc_call = pl.kernel(
    _body,
    out_type=[
        jax.ShapeDtypeStruct((B,), jnp.float32),
        jax.ShapeDtypeStruct((B,), jnp.float32),
        jax.ShapeDtypeStruct((NW, L), jnp.float32),
    ],
    mesh=_mesh,
    compiler_params=pltpu.CompilerParams(needs_layout_passes=False),
    scratch_types=[
        pltpu.VMEM((BW,), jnp.int32),
        pltpu.VMEM((BW,), jnp.int32),
        pltpu.VMEM((BW,), jnp.int32),
        pltpu.VMEM((C, D), jnp.float32),
        pltpu.VMEM((C, D), jnp.float32),
        pltpu.VMEM((C, D), jnp.float32),
        pltpu.VMEM((C, D), jnp.float32),
        pltpu.VMEM((C, D), jnp.float32),
        pltpu.VMEM((C, D), jnp.float32),
        pltpu.VMEM((L, L), jnp.float32),
        pltpu.VMEM((L, L), jnp.float32),
        pltpu.VMEM((BW,), jnp.float32),
        pltpu.VMEM((BW,), jnp.float32),
        pltpu.VMEM((L,), jnp.float32),
        pltpu.SemaphoreType.DMA,
        pltpu.SemaphoreType.DMA,
        pltpu.SemaphoreType.DMA,
    ],
)


def kernel(user_idx, pos_item, neg_item, user_table, item_table):
    ui = user_idx.astype(jnp.int32)
    pi = pos_item.astype(jnp.int32)
    ni = neg_item.astype(jnp.int32)
    pos_scores, neg_scores, reg_part = _sc_call(ui, pi, ni, user_table, item_table)
    return pos_scores, neg_scores, jnp.sum(reg_part)
